# Initial kernel scaffold; baseline (speedup 1.0000x reference)
import jax
import jax.numpy as jnp
from jax.experimental import pallas as pl


def kernel(non_zero, input, W, a):
    raise NotImplementedError("write your pallas kernel here")


# speedup vs baseline: 4.8413x; 4.8413x over previous
import jax
import jax.numpy as jnp
from jax import lax
from jax.experimental import pallas as pl
from jax.experimental.pallas import tpu as pltpu
from jax.experimental.pallas import tpu_sc as plsc

N = 10000
E = 320000
F = 128

NC = 2
NS = 16
FH = F // NC
E_PER = E // NS
CH = 80
NCH = E_PER // CH
ROWS_PER_TILE = N // NS


def _tc_pre_body(x_ref, w_ref, a1_ref, a2_ref, h_ref, f1_ref, f2_ref):
    h = jnp.dot(x_ref[...], w_ref[...], preferred_element_type=jnp.float32)
    h_ref[...] = h
    f1_ref[...] = jnp.dot(h, a1_ref[...], preferred_element_type=jnp.float32)
    f2_ref[...] = jnp.dot(h, a2_ref[...], preferred_element_type=jnp.float32)


_tc_pre = pl.pallas_call(
    _tc_pre_body,
    out_shape=[
        jax.ShapeDtypeStruct((N, F), jnp.float32),
        jax.ShapeDtypeStruct((N, 1), jnp.float32),
        jax.ShapeDtypeStruct((N, 1), jnp.float32),
    ],
)


def _sc_body(src2_hbm, dst2_hbm, f1_hbm, f2_hbm, h2_hbm, zeros_hbm,
             ee_hbm, part_hbm,
             src2_v, dst2_v, w2_v, f1_v, f2_v, rows_v, shared, sem):
    c = lax.axis_index("c")
    s = lax.axis_index("s")

    pltpu.sync_copy(src2_hbm.at[s], src2_v)
    pltpu.sync_copy(dst2_hbm.at[s], dst2_v)
    pltpu.sync_copy(f1_hbm, f1_v)
    pltpu.sync_copy(f2_hbm, f2_v)

    pltpu.sync_copy(zeros_hbm.at[s],
                    shared.at[pl.ds(s * ROWS_PER_TILE, ROWS_PER_TILE)])

    def wbody(ci, _):
        for k in range(CH // 16):
            si = src2_v[ci, pl.ds(k * 16, 16)]
            di = dst2_v[ci, pl.ds(k * 16, 16)]
            sc = plsc.load_gather(f1_v, [si]) + plsc.load_gather(f2_v, [di])
            lr = jnp.where(sc >= 0.0, sc, sc * 0.2)
            w2_v[ci, pl.ds(k * 16, 16)] = jnp.exp(-lr)
        return 0

    lax.fori_loop(0, NCH, wbody, 0)

    @pl.when(c == 0)
    def _():
        pltpu.sync_copy(w2_v, ee_hbm.at[s])

    plsc.subcore_barrier()

    hview = h2_hbm.at[c]

    def mbody(ci, _):
        pltpu.async_copy(hview.at[dst2_v.at[ci]], rows_v, sem).wait()

        def ebody(k, _):
            wv = plsc.load_gather(
                w2_v,
                [jnp.full((16,), ci, jnp.int32),
                 jnp.full((16,), k, jnp.int32)])
            for j in range(FH // 16):
                rows_v[k, pl.ds(j * 16, 16)] = (
                    rows_v[k, pl.ds(j * 16, 16)] * wv)
            return 0

        lax.fori_loop(0, CH, ebody, 0)
        pltpu.sync_copy(rows_v, shared.at[src2_v.at[ci]], add=True)
        return 0

    lax.fori_loop(0, NCH, mbody, 0)
    plsc.subcore_barrier()

    pltpu.sync_copy(shared.at[pl.ds(s * ROWS_PER_TILE, ROWS_PER_TILE)],
                    part_hbm.at[c, s])


_sc_edge = pl.kernel(
    _sc_body,
    out_type=[
        jax.ShapeDtypeStruct((NS, NCH, CH), jnp.float32),
        jax.ShapeDtypeStruct((NC, NS, ROWS_PER_TILE, FH), jnp.float32),
    ],
    mesh=plsc.VectorSubcoreMesh(core_axis_name="c", subcore_axis_name="s"),
    compiler_params=pltpu.CompilerParams(
        needs_layout_passes=False, use_tc_tiling_on_sc=False),
    scratch_types=[
        pltpu.VMEM((NCH, CH), jnp.int32),
        pltpu.VMEM((NCH, CH), jnp.int32),
        pltpu.VMEM((NCH, CH), jnp.float32),
        pltpu.VMEM((N,), jnp.float32),
        pltpu.VMEM((N,), jnp.float32),
        pltpu.VMEM((CH, FH), jnp.float32),
        pltpu.VMEM_SHARED((N, FH), jnp.float32),
        pltpu.SemaphoreType.DMA,
    ],
)


def _tc_post_body(p0_ref, p1_ref, o_ref):
    x = jnp.concatenate([p0_ref[...], p1_ref[...]], axis=1)
    o_ref[...] = jnp.where(x > 0.0, x, jnp.exp(x) - 1.0)


_tc_post = pl.pallas_call(
    _tc_post_body,
    out_shape=jax.ShapeDtypeStruct((N, F), jnp.float32),
)


def kernel(non_zero, input, W, a):
    src = non_zero[0, :]
    dst = non_zero[1, :]
    a1 = a[0, :F].reshape(F, 1)
    a2 = a[0, F:].reshape(F, 1)
    h, f1, f2 = _tc_pre(input, W, a1, a2)
    h2 = jnp.stack([h[:, :FH], h[:, FH:]])
    src2 = src.reshape(NS, NCH, CH)
    dst2 = dst.reshape(NS, NCH, CH)
    zeros = jnp.zeros((NS, ROWS_PER_TILE, FH), jnp.float32)
    ee, part = _sc_edge(src2, dst2, f1.reshape(N), f2.reshape(N), h2, zeros)
    out = _tc_post(part[0].reshape(N, FH), part[1].reshape(N, FH))
    return out, ee.reshape(E)


# speedup vs baseline: 9.0782x; 1.8752x over previous
import jax
import jax.numpy as jnp
from jax import lax
from jax.experimental import pallas as pl
from jax.experimental.pallas import tpu as pltpu
from jax.experimental.pallas import tpu_sc as plsc

N = 10000
E = 320000
F = 128

NC = 2
NS = 16
FH = F // NC
E_PER = E // NS
CH = 80
NCH = E_PER // CH
ROWS_PER_TILE = N // NS


def _tc_pre_body(x_ref, w_ref, a1_ref, a2_ref, h_ref, f1_ref, f2_ref):
    h = jnp.dot(x_ref[...], w_ref[...], preferred_element_type=jnp.float32)
    h_ref[...] = h
    f1_ref[...] = jnp.dot(h, a1_ref[...], preferred_element_type=jnp.float32)
    f2_ref[...] = jnp.dot(h, a2_ref[...], preferred_element_type=jnp.float32)


_tc_pre = pl.pallas_call(
    _tc_pre_body,
    out_shape=[
        jax.ShapeDtypeStruct((N, F), jnp.float32),
        jax.ShapeDtypeStruct((N, 1), jnp.float32),
        jax.ShapeDtypeStruct((N, 1), jnp.float32),
    ],
)


def _sc_body(src2_hbm, dst2_hbm, f1_hbm, f2_hbm, h2_hbm, zeros_hbm,
             ee_hbm, part_hbm,
             src2_v, dst2_v, w2_v, f1_v, f2_v, rows_a, rows_b, shared,
             gsem_a, gsem_b, ssem_a, ssem_b):
    c = lax.axis_index("c")
    s = lax.axis_index("s")

    pltpu.sync_copy(src2_hbm.at[s], src2_v)
    pltpu.sync_copy(dst2_hbm.at[s], dst2_v)
    pltpu.sync_copy(f1_hbm, f1_v)
    pltpu.sync_copy(f2_hbm, f2_v)

    pltpu.sync_copy(zeros_hbm.at[s],
                    shared.at[pl.ds(s * ROWS_PER_TILE, ROWS_PER_TILE)])

    @plsc.parallel_loop(0, NCH, unroll=2)
    def wbody(ci):
        for k in range(CH // 16):
            si = src2_v[ci, pl.ds(k * 16, 16)]
            di = dst2_v[ci, pl.ds(k * 16, 16)]
            sc = plsc.load_gather(f1_v, [si]) + plsc.load_gather(f2_v, [di])
            lr = jnp.where(sc >= 0.0, sc, sc * 0.2)
            w2_v[ci, pl.ds(k * 16, 16)] = jnp.exp(-lr)

    @pl.when(c == 0)
    def _():
        pltpu.sync_copy(w2_v, ee_hbm.at[s])

    plsc.subcore_barrier()

    hview = h2_hbm.at[c]

    def scale(rows, ci):
        @plsc.parallel_loop(0, CH, unroll=8)
        def ebody(k):
            wv = plsc.load_gather(
                w2_v,
                [jnp.full((16,), ci, jnp.int32),
                 jnp.full((16,), k, jnp.int32)])
            for j in range(FH // 16):
                rows[k, pl.ds(j * 16, 16)] = rows[k, pl.ds(j * 16, 16)] * wv

    pltpu.async_copy(hview.at[dst2_v.at[0]], rows_a, gsem_a)

    def mbody(cio, _):
        ci = cio * 2
        pltpu.async_copy(hview.at[dst2_v.at[ci + 1]], rows_b, gsem_b)
        pltpu.make_async_copy(hview.at[dst2_v.at[ci]], rows_a, gsem_a).wait()
        scale(rows_a, ci)
        pltpu.async_copy(rows_a, shared.at[src2_v.at[ci]], ssem_a, add=True)
        pltpu.make_async_copy(hview.at[dst2_v.at[ci + 1]], rows_b,
                              gsem_b).wait()
        scale(rows_b, ci + 1)
        pltpu.async_copy(rows_b, shared.at[src2_v.at[ci + 1]], ssem_b,
                         add=True)
        pltpu.make_async_copy(rows_a, shared.at[src2_v.at[ci]], ssem_a).wait()

        @pl.when(ci + 2 < NCH)
        def _():
            pltpu.async_copy(hview.at[dst2_v.at[ci + 2]], rows_a, gsem_a)

        pltpu.make_async_copy(rows_b, shared.at[src2_v.at[ci + 1]],
                              ssem_b).wait()
        return 0

    lax.fori_loop(0, NCH // 2, mbody, 0)
    plsc.subcore_barrier()

    pltpu.sync_copy(shared.at[pl.ds(s * ROWS_PER_TILE, ROWS_PER_TILE)],
                    part_hbm.at[c, s])


_sc_edge = pl.kernel(
    _sc_body,
    out_type=[
        jax.ShapeDtypeStruct((NS, NCH, CH), jnp.float32),
        jax.ShapeDtypeStruct((NC, NS, ROWS_PER_TILE, FH), jnp.float32),
    ],
    mesh=plsc.VectorSubcoreMesh(core_axis_name="c", subcore_axis_name="s"),
    compiler_params=pltpu.CompilerParams(
        needs_layout_passes=False, use_tc_tiling_on_sc=False),
    scratch_types=[
        pltpu.VMEM((NCH, CH), jnp.int32),
        pltpu.VMEM((NCH, CH), jnp.int32),
        pltpu.VMEM((NCH, CH), jnp.float32),
        pltpu.VMEM((N,), jnp.float32),
        pltpu.VMEM((N,), jnp.float32),
        pltpu.VMEM((CH, FH), jnp.float32),
        pltpu.VMEM((CH, FH), jnp.float32),
        pltpu.VMEM_SHARED((N, FH), jnp.float32),
        pltpu.SemaphoreType.DMA,
        pltpu.SemaphoreType.DMA,
        pltpu.SemaphoreType.DMA,
        pltpu.SemaphoreType.DMA,
    ],
)


def _tc_post_body(p0_ref, p1_ref, o_ref):
    x = jnp.concatenate([p0_ref[...], p1_ref[...]], axis=1)
    o_ref[...] = jnp.where(x > 0.0, x, jnp.exp(x) - 1.0)


_tc_post = pl.pallas_call(
    _tc_post_body,
    out_shape=jax.ShapeDtypeStruct((N, F), jnp.float32),
)


def kernel(non_zero, input, W, a):
    src = non_zero[0, :]
    dst = non_zero[1, :]
    a1 = a[0, :F].reshape(F, 1)
    a2 = a[0, F:].reshape(F, 1)
    h, f1, f2 = _tc_pre(input, W, a1, a2)
    h2 = jnp.stack([h[:, :FH], h[:, FH:]])
    src2 = src.reshape(NS, NCH, CH)
    dst2 = dst.reshape(NS, NCH, CH)
    zeros = jnp.zeros((NS, ROWS_PER_TILE, FH), jnp.float32)
    ee, part = _sc_edge(src2, dst2, f1.reshape(N), f2.reshape(N), h2, zeros)
    out = _tc_post(part[0].reshape(N, FH), part[1].reshape(N, FH))
    return out, ee.reshape(E)
